# baseline (speedup 1.0000x reference)
import jax
import jax.numpy as jnp
from jax.experimental import pallas as pl

NUM_EMBEDDINGS = 1024
EMBEDDING_DIM = 64
COMMITMENT_COST = 0.25
BM = 512


def _vq_body(z_ref, e_ref, idx_ref, zq_ref, loss_ref):
    i = pl.program_id(0)
    z = z_ref[...]
    e = e_ref[...]
    mm = jax.lax.dot_general(z, e, (((1,), (1,)), ((), ())),
                             preferred_element_type=jnp.float32)
    zn = jnp.sum(z * z, axis=1, keepdims=True)
    en = jnp.sum(e * e, axis=1)[None, :]
    d = (zn + en) - 2.0 * mm
    minv = jnp.min(d, axis=1, keepdims=True)
    iota = jax.lax.broadcasted_iota(jnp.int32, d.shape, 1)
    idx = jnp.min(jnp.where(d == minv, iota, NUM_EMBEDDINGS), axis=1)
    idx_ref[...] = idx
    oh = (iota == idx[:, None]).astype(jnp.float32)
    zq = jax.lax.dot_general(oh, e, (((1,), (0,)), ((), ())),
                             preferred_element_type=jnp.float32,
                             precision=jax.lax.Precision.HIGHEST)
    zq_ref[...] = z + (zq - z)
    bs = jnp.sum((z - zq) ** 2)[None, None]

    @pl.when(i == 0)
    def _init():
        loss_ref[...] = jnp.zeros_like(loss_ref)

    loss_ref[...] += bs


def kernel(z_e, embedding):
    B, T, C = z_e.shape
    M = B * T
    z_flat = z_e.reshape(M, C)
    grid = (M // BM,)
    idx_out, zq_out, loss_out = pl.pallas_call(
        _vq_body,
        grid=grid,
        in_specs=[
            pl.BlockSpec((BM, C), lambda i: (i, 0)),
            pl.BlockSpec((NUM_EMBEDDINGS, C), lambda i: (0, 0)),
        ],
        out_specs=[
            pl.BlockSpec((BM,), lambda i: (i,)),
            pl.BlockSpec((BM, C), lambda i: (i, 0)),
            pl.BlockSpec((1, 1), lambda i: (0, 0)),
        ],
        out_shape=[
            jax.ShapeDtypeStruct((M,), jnp.int32),
            jax.ShapeDtypeStruct((M, C), jnp.float32),
            jax.ShapeDtypeStruct((1, 1), jnp.float32),
        ],
    )(z_flat, embedding)
    commitment_loss = (COMMITMENT_COST / (M * C)) * loss_out[0, 0]
    return (zq_out.reshape(B, T, C), commitment_loss, idx_out.reshape(B, T))


# speedup vs baseline: 1.2503x; 1.2503x over previous
import functools

import jax
import jax.numpy as jnp
from jax import lax
from jax.experimental import pallas as pl
from jax.experimental.pallas import tpu as pltpu
from jax.experimental.pallas import tpu_sc as plsc

NUM_EMBEDDINGS = 1024
EMBEDDING_DIM = 64
COMMITMENT_COST = 0.25
BM = 512

_NC, _NS = 2, 16
_NW = _NC * _NS


def _vq_body(z_ref, e_ref, idx_ref, loss_ref):
    i = pl.program_id(0)
    z = z_ref[...]
    e = e_ref[...]
    mm = lax.dot_general(z, e, (((1,), (1,)), ((), ())),
                         preferred_element_type=jnp.float32)
    zn = jnp.sum(z * z, axis=1, keepdims=True)
    en = jnp.sum(e * e, axis=1)[None, :]
    d = (zn + en) - 2.0 * mm
    minv = jnp.min(d, axis=1, keepdims=True)
    iota = lax.broadcasted_iota(jnp.int32, d.shape, 1)
    idx = jnp.min(jnp.where(d == minv, iota, NUM_EMBEDDINGS), axis=1)
    idx_ref[...] = idx
    bs = jnp.sum(minv)[None, None]

    @pl.when(i == 0)
    def _init():
        loss_ref[...] = jnp.zeros_like(loss_ref)

    loss_ref[...] += bs


def _make_sc_gather(V, D, B):
    b_per_w = B // _NW
    mesh = plsc.VectorSubcoreMesh(core_axis_name="c", subcore_axis_name="s")

    @functools.partial(
        pl.kernel, mesh=mesh,
        out_type=jax.ShapeDtypeStruct((B, 128), jnp.float32),
        scratch_types=[
            pltpu.VMEM((b_per_w,), jnp.int32),
            pltpu.VMEM((b_per_w, 128), jnp.float32),
            pltpu.SemaphoreType.DMA,
        ],
    )
    def gather_k(table_hbm, idx_hbm, out_hbm, idx_v, rows_v, sem):
        wid = lax.axis_index("s") * _NC + lax.axis_index("c")
        base = wid * b_per_w
        pltpu.sync_copy(idx_hbm.at[pl.ds(base, b_per_w)], idx_v)
        pltpu.async_copy(table_hbm.at[idx_v], rows_v, sem).wait()
        pltpu.sync_copy(rows_v, out_hbm.at[pl.ds(base, b_per_w)])

    return gather_k


def kernel(z_e, embedding):
    B, T, C = z_e.shape
    M = B * T
    z_flat = z_e.reshape(M, C)
    grid = (M // BM,)
    idx_out, loss_out = pl.pallas_call(
        _vq_body,
        grid=grid,
        in_specs=[
            pl.BlockSpec((BM, C), lambda i: (i, 0)),
            pl.BlockSpec((NUM_EMBEDDINGS, C), lambda i: (0, 0)),
        ],
        out_specs=[
            pl.BlockSpec((BM,), lambda i: (i,)),
            pl.BlockSpec((1, 1), lambda i: (0, 0)),
        ],
        out_shape=[
            jax.ShapeDtypeStruct((M,), jnp.int32),
            jax.ShapeDtypeStruct((1, 1), jnp.float32),
        ],
    )(z_flat, embedding)
    emb_pad = jnp.pad(embedding, ((0, 0), (0, 128 - C)))
    zq128 = _make_sc_gather(NUM_EMBEDDINGS, C, M)(emb_pad, idx_out)
    zq = zq128[:, :C]
    commitment_loss = (COMMITMENT_COST / (M * C)) * loss_out[0, 0]
    return (zq.reshape(B, T, C), commitment_loss, idx_out.reshape(B, T))


# speedup vs baseline: 1.3003x; 1.0400x over previous
import functools

import jax
import jax.numpy as jnp
from jax import lax
from jax.experimental import pallas as pl
from jax.experimental.pallas import tpu as pltpu
from jax.experimental.pallas import tpu_sc as plsc

NUM_EMBEDDINGS = 1024
EMBEDDING_DIM = 64
COMMITMENT_COST = 0.25
BM = 1024

_NC, _NS = 2, 16
_NW = _NC * _NS


def _vq_body(z_ref, zn_ref, en_ref, e_ref, idx_ref, loss_ref):
    i = pl.program_id(0)
    z = z_ref[...]
    e = e_ref[...]
    mm2 = lax.dot_general(-2.0 * z, e, (((1,), (1,)), ((), ())),
                          preferred_element_type=jnp.float32)
    t = zn_ref[...] + en_ref[...]
    d = t + mm2
    minv = jnp.min(d, axis=1, keepdims=True)
    iota_f = lax.broadcasted_iota(jnp.int32, d.shape, 1).astype(jnp.float32)
    big = jnp.float32(NUM_EMBEDDINGS)
    idxf = jnp.min(jnp.where(d == minv, iota_f, big), axis=1)
    idx_ref[...] = idxf.astype(jnp.int32)
    bs = jnp.sum(minv)[None, None]

    @pl.when(i == 0)
    def _init():
        loss_ref[...] = jnp.zeros_like(loss_ref)

    loss_ref[...] += bs


def _make_sc_gather(V, D, B):
    b_per_w = B // _NW
    mesh = plsc.VectorSubcoreMesh(core_axis_name="c", subcore_axis_name="s")

    @functools.partial(
        pl.kernel, mesh=mesh,
        out_type=jax.ShapeDtypeStruct((B, 128), jnp.float32),
        scratch_types=[
            pltpu.VMEM((b_per_w,), jnp.int32),
            pltpu.VMEM((b_per_w, 128), jnp.float32),
            pltpu.SemaphoreType.DMA,
        ],
    )
    def gather_k(table_hbm, idx_hbm, out_hbm, idx_v, rows_v, sem):
        wid = lax.axis_index("s") * _NC + lax.axis_index("c")
        base = wid * b_per_w
        pltpu.sync_copy(idx_hbm.at[pl.ds(base, b_per_w)], idx_v)
        pltpu.async_copy(table_hbm.at[idx_v], rows_v, sem).wait()
        pltpu.sync_copy(rows_v, out_hbm.at[pl.ds(base, b_per_w)])

    return gather_k


def kernel(z_e, embedding):
    B, T, C = z_e.shape
    M = B * T
    z_flat = z_e.reshape(M, C)
    zn = jnp.sum(z_flat ** 2, axis=1, keepdims=True)
    en = jnp.sum(embedding ** 2, axis=1, keepdims=True).T
    grid = (M // BM,)
    idx_out, loss_out = pl.pallas_call(
        _vq_body,
        grid=grid,
        in_specs=[
            pl.BlockSpec((BM, C), lambda i: (i, 0)),
            pl.BlockSpec((BM, 1), lambda i: (i, 0)),
            pl.BlockSpec((1, NUM_EMBEDDINGS), lambda i: (0, 0)),
            pl.BlockSpec((NUM_EMBEDDINGS, C), lambda i: (0, 0)),
        ],
        out_specs=[
            pl.BlockSpec((BM,), lambda i: (i,)),
            pl.BlockSpec((1, 1), lambda i: (0, 0)),
        ],
        out_shape=[
            jax.ShapeDtypeStruct((M,), jnp.int32),
            jax.ShapeDtypeStruct((1, 1), jnp.float32),
        ],
    )(z_flat, zn, en, embedding)
    emb_pad = jnp.pad(embedding, ((0, 0), (0, 128 - C)))
    zq128 = _make_sc_gather(NUM_EMBEDDINGS, C, M)(emb_pad, idx_out)
    zq = zq128[:, :C]
    commitment_loss = (COMMITMENT_COST / (M * C)) * loss_out[0, 0]
    return (zq.reshape(B, T, C), commitment_loss, idx_out.reshape(B, T))
